# Initial kernel scaffold; baseline (speedup 1.0000x reference)
import jax
import jax.numpy as jnp
from jax.experimental import pallas as pl


def kernel(x, edge_index, Wn0, bn0, Wr0, br0, Wn1, bn1, Wr1, br1, Wn2, bn2, Wr2, br2, g0, be0, rm0, rv0, g1, be1, rm1, rv1):
    raise NotImplementedError("write your pallas kernel here")


# speedup vs baseline: 4.0744x; 4.0744x over previous
import functools

import jax
import jax.numpy as jnp
from jax import lax
from jax.experimental import pallas as pl
from jax.experimental.pallas import tpu as pltpu
from jax.experimental.pallas import tpu_sc as plsc

N = 10000
E = 320000
D = 128
TAU = 0.5

NC = 2
NS = 16
NW = NC * NS
K = 80
EPW = E // NW
NCH1 = EPW // K
EPS = E // NS
NCH2 = EPS // K
HD = N // NC
SPR = 5056
DB = 5008
LCAP = EPS + 96
ZL = 104
RS = 312
BM = 400

_mesh = plsc.VectorSubcoreMesh(core_axis_name="c", subcore_axis_name="s")
_sc_params = pltpu.CompilerParams(needs_layout_passes=False)


def _ew_body(xn, src, dst, el_out,
             src_v, dst_v, a_v, b_v, el_v, tile_v, sem_a, sem_b):
    c = lax.axis_index("c")
    s = lax.axis_index("s")
    wid = s * NC + c

    def chunk(i, _):
        base = wid * EPW + i * K
        pltpu.sync_copy(src.at[pl.ds(base, K)], src_v)
        pltpu.sync_copy(dst.at[pl.ds(base, K)], dst_v)
        cp_a = pltpu.async_copy(xn.at[src_v], a_v, sem_a)
        cp_b = pltpu.async_copy(xn.at[dst_v], b_v, sem_b)
        cp_a.wait()
        cp_b.wait()

        ioff = lax.iota(jnp.int32, 16) * 16
        for g in range(K // 16):
            for e2 in range(16):
                e = g * 16 + e2
                acc = a_v[e, pl.ds(0, 16)] * b_v[e, pl.ds(0, 16)]
                for j in range(1, 8):
                    acc = acc + (a_v[e, pl.ds(16 * j, 16)]
                                 * b_v[e, pl.ds(16 * j, 16)])
                tile_v[pl.ds(e2 * 16, 16)] = acc
            sims = plsc.load_gather(tile_v, [ioff])
            for l in range(1, 16):
                sims = sims + plsc.load_gather(tile_v, [ioff + l])
            el_v[pl.ds(16 * g, 16)] = jnp.exp(sims * (1.0 / TAU))

        pltpu.sync_copy(el_v, el_out.at[pl.ds(base, K)])
        return 0
    lax.fori_loop(0, NCH1, chunk, 0)


_ew_call = pl.kernel(
    _ew_body,
    out_type=jax.ShapeDtypeStruct((E,), jnp.float32),
    mesh=_mesh,
    compiler_params=_sc_params,
    scratch_types=[
        pltpu.VMEM((K,), jnp.int32),
        pltpu.VMEM((K,), jnp.int32),
        pltpu.VMEM((K, D), jnp.float32),
        pltpu.VMEM((K, D), jnp.float32),
        pltpu.VMEM((K,), jnp.float32),
        pltpu.VMEM((256,), jnp.float32),
        pltpu.SemaphoreType.DMA,
        pltpu.SemaphoreType.DMA,
    ],
)


def _part_body(src, dst, el, lsrc, ldst, lel, cnts,
               src_v, dst_v, el_v, ssrc, sdst, sel, cnt_v):
    c = lax.axis_index("c")
    s = lax.axis_index("s")
    lo = c * HD

    def chunk(i, wp):
        base = s * EPS + i * K
        pltpu.sync_copy(src.at[pl.ds(base, K)], src_v)
        pltpu.sync_copy(dst.at[pl.ds(base, K)], dst_v)
        pltpu.sync_copy(el.at[pl.ds(base, K)], el_v)
        for g in range(K // 16):
            sl = pl.ds(g * 16, 16)
            d16 = dst_v[sl]
            mask = (d16 >= lo) & (d16 < lo + HD)
            plsc.store_compressed(ssrc.at[pl.ds(wp, 16)], src_v[sl], mask=mask)
            plsc.store_compressed(sdst.at[pl.ds(wp, 16)], d16 - lo, mask=mask)
            plsc.store_compressed(sel.at[pl.ds(wp, 16)], el_v[sl], mask=mask)
            wp = wp + plsc.all_reduce_population_count(mask)[0]
        return wp
    wp = lax.fori_loop(0, NCH2, chunk, jnp.int32(0))

    def padz(i, _):
        off = pl.ds(wp + i * 16, 16)
        ssrc[off] = jnp.zeros((16,), jnp.int32)
        sdst[off] = jnp.zeros((16,), jnp.int32)
        sel[off] = jnp.zeros((16,), jnp.float32)
        return 0
    lax.fori_loop(0, 6, padz, 0)

    rbase = (c * NS + s) * LCAP
    pltpu.sync_copy(ssrc, lsrc.at[pl.ds(rbase, LCAP)])
    pltpu.sync_copy(sdst, ldst.at[pl.ds(rbase, LCAP)])
    pltpu.sync_copy(sel, lel.at[pl.ds(rbase, LCAP)])
    cnt_v[:] = jnp.full((16,), wp, jnp.int32)
    pltpu.sync_copy(cnt_v, cnts.at[pl.ds((c * NS + s) * 16, 16)])


_part_call = pl.kernel(
    _part_body,
    out_type=(jax.ShapeDtypeStruct((NC * NS * LCAP,), jnp.int32),
              jax.ShapeDtypeStruct((NC * NS * LCAP,), jnp.int32),
              jax.ShapeDtypeStruct((NC * NS * LCAP,), jnp.float32),
              jax.ShapeDtypeStruct((NC * NS * 16,), jnp.int32)),
    mesh=_mesh,
    compiler_params=_sc_params,
    scratch_types=[
        pltpu.VMEM((K,), jnp.int32),
        pltpu.VMEM((K,), jnp.int32),
        pltpu.VMEM((K,), jnp.float32),
        pltpu.VMEM((LCAP,), jnp.int32),
        pltpu.VMEM((LCAP,), jnp.int32),
        pltpu.VMEM((LCAP,), jnp.float32),
        pltpu.VMEM((16,), jnp.int32),
    ],
)


def _agg_body(h, lsrc, ldst, lel, cnts, un_out, den_out,
              src_v, dst_v, el_v, rows_v, zbuf_v, cnts_v, den_v, didx_v,
              agg_sp, sem, *, with_den):
    c = lax.axis_index("c")
    s = lax.axis_index("s")

    def zf(r, _):
        for j in range(8):
            zbuf_v[r, pl.ds(16 * j, 16)] = jnp.zeros((16,), jnp.float32)
        return 0
    lax.fori_loop(0, ZL, zf, 0)
    for t in range(RS // ZL):
        pltpu.sync_copy(zbuf_v, agg_sp.at[pl.ds(s * RS + t * ZL, ZL)])

    @pl.when(s == NS - 1)
    def _():
        pltpu.sync_copy(zbuf_v.at[pl.ds(0, SPR - NS * RS)],
                        agg_sp.at[pl.ds(NS * RS, SPR - NS * RS)])

    if with_den:
        def zd(r, _):
            for j in range(8):
                den_v[r, pl.ds(16 * j, 16)] = jnp.zeros((16,), jnp.float32)
            return 0
        lax.fori_loop(0, 48, zd, 0)
        ioz = lax.iota(jnp.int32, 16)
        for g in range(3):
            didx_v[pl.ds(g * 16, 16)] = ioz + (DB + g * 16)
    plsc.subcore_barrier()

    pltpu.sync_copy(cnts.at[pl.ds((c * NS + s) * 16, 16)], cnts_v)
    cnt = cnts_v[pl.ds(0, 16)][0]
    nch = (cnt + (K - 1)) // K
    rbase = (c * NS + s) * LCAP

    def chunk(i, _):
        base = rbase + i * K
        pltpu.sync_copy(lsrc.at[pl.ds(base, K)], src_v)
        pltpu.sync_copy(ldst.at[pl.ds(base, K)], dst_v)
        pltpu.sync_copy(lel.at[pl.ds(base, K)], el_v)
        pltpu.async_copy(h.at[src_v], rows_v, sem).wait()

        ioz16 = lax.iota(jnp.int32, 16)
        for g in range(K // 16):
            el16 = el_v[pl.ds(g * 16, 16)]
            if with_den:
                d16 = dst_v[pl.ds(g * 16, 16)]
            for e2 in range(16):
                e = g * 16 + e2
                wv = jnp.full((16,), el16[e2], jnp.float32)
                for j in range(8):
                    rows_v[e, pl.ds(16 * j, 16)] = (
                        rows_v[e, pl.ds(16 * j, 16)] * wv)
                if with_den:
                    ld = d16[e2]
                    row = lax.shift_right_logical(ld, 7)
                    off = (lax.shift_right_logical(ld, 4) & 7) * 16
                    lane = ld & 15
                    m = ioz16 == jnp.full((16,), lane, jnp.int32)
                    addv = jnp.where(m, wv, jnp.zeros((16,), jnp.float32))
                    den_v[row, pl.ds(off, 16)] = (
                        den_v[row, pl.ds(off, 16)] + addv)

        pltpu.sync_copy(rows_v, agg_sp.at[dst_v], add=True)
        return 0
    lax.fori_loop(0, nch, chunk, 0)

    if with_den:
        pltpu.sync_copy(den_v, agg_sp.at[didx_v], add=True)

    plsc.subcore_barrier()
    for t in range(RS // ZL):
        sl = pl.ds(s * RS + t * ZL, ZL)
        pltpu.sync_copy(agg_sp.at[sl], zbuf_v)
        pltpu.sync_copy(zbuf_v, un_out.at[c, sl])

    @pl.when(s == NS - 1)
    def _():
        tl = pl.ds(NS * RS, HD - NS * RS)
        pltpu.sync_copy(agg_sp.at[tl], zbuf_v.at[pl.ds(0, HD - NS * RS)])
        pltpu.sync_copy(zbuf_v.at[pl.ds(0, HD - NS * RS)], un_out.at[c, tl])

    if with_den:
        @pl.when(s == 0)
        def _():
            dl = pl.ds(DB, 40)
            pltpu.sync_copy(agg_sp.at[dl], zbuf_v.at[pl.ds(0, 40)])
            pltpu.sync_copy(zbuf_v.at[pl.ds(0, 40)], den_out.at[c])


def _make_agg_call(with_den):
    return pl.kernel(
        functools.partial(_agg_body, with_den=with_den),
        out_type=(jax.ShapeDtypeStruct((NC, HD, D), jnp.float32),
                  jax.ShapeDtypeStruct((NC, 40, D), jnp.float32)),
        mesh=_mesh,
        compiler_params=_sc_params,
        scratch_types=[
            pltpu.VMEM((K,), jnp.int32),
            pltpu.VMEM((K,), jnp.int32),
            pltpu.VMEM((K,), jnp.float32),
            pltpu.VMEM((K, D), jnp.float32),
            pltpu.VMEM((ZL, D), jnp.float32),
            pltpu.VMEM((16,), jnp.int32),
            pltpu.VMEM((48, D), jnp.float32),
            pltpu.VMEM((48,), jnp.int32),
            pltpu.VMEM_SHARED((SPR, D), jnp.float32),
            pltpu.SemaphoreType.DMA,
        ],
    )


_agg_den_call = _make_agg_call(True)
_agg_call = _make_agg_call(False)


def _norm_body(x_ref, o_ref):
    x = x_ref[...]
    n = jnp.sqrt(jnp.sum(x * x, axis=1, keepdims=True))
    o_ref[...] = x / (n + 1e-12)


_norm_call = pl.pallas_call(
    _norm_body,
    grid=(N // BM,),
    in_specs=[pl.BlockSpec((BM, D), lambda i: (i, 0))],
    out_specs=pl.BlockSpec((BM, D), lambda i: (i, 0)),
    out_shape=jax.ShapeDtypeStruct((N, D), jnp.float32),
)


def _layer_body(un_ref, den_ref, h_ref, wn_ref, bn_ref, wr_ref, br_ref,
                g_ref, be_ref, rm_ref, rv_ref, o_ref, *, with_bn):
    den = den_ref[...]
    agg = un_ref[...] / (den + 1e-16)
    h = h_ref[...]
    y = lax.dot_general(agg, wn_ref[...], (((1,), (1,)), ((), ())),
                        preferred_element_type=jnp.float32)
    y = y + lax.dot_general(h, wr_ref[...], (((1,), (1,)), ((), ())),
                            preferred_element_type=jnp.float32)
    y = y + bn_ref[...] + br_ref[...]
    if with_bn:
        y = (y - rm_ref[...]) * lax.rsqrt(rv_ref[...] + 1e-5) * g_ref[...]
        y = jnp.maximum(y + be_ref[...], 0.0)
    o_ref[...] = y


def _make_layer_call(with_bn):
    full = lambda i: (0, 0)
    return pl.pallas_call(
        functools.partial(_layer_body, with_bn=with_bn),
        grid=(N // BM,),
        in_specs=[
            pl.BlockSpec((BM, D), lambda i: (i, 0)),
            pl.BlockSpec((BM, 1), lambda i: (i, 0)),
            pl.BlockSpec((BM, D), lambda i: (i, 0)),
            pl.BlockSpec((D, D), full),
            pl.BlockSpec((1, D), full),
            pl.BlockSpec((D, D), full),
            pl.BlockSpec((1, D), full),
            pl.BlockSpec((1, D), full),
            pl.BlockSpec((1, D), full),
            pl.BlockSpec((1, D), full),
            pl.BlockSpec((1, D), full),
        ],
        out_specs=pl.BlockSpec((BM, D), lambda i: (i, 0)),
        out_shape=jax.ShapeDtypeStruct((N, D), jnp.float32),
    )


_layer_bn_call = _make_layer_call(True)
_layer_plain_call = _make_layer_call(False)


def kernel(x, edge_index, Wn0, bn0, Wr0, br0, Wn1, bn1, Wr1, br1,
           Wn2, bn2, Wr2, br2, g0, be0, rm0, rv0, g1, be1, rm1, rv1):
    src = edge_index[0]
    dst = edge_index[1]
    xn = _norm_call(x)
    el = _ew_call(xn, src, dst)
    lsrc, ldst, lel, cnts = _part_call(src, dst, el)

    r2 = lambda v: v.reshape(1, D)
    h = x
    den_col = None
    params = [
        (Wn0, bn0, Wr0, br0, g0, be0, rm0, rv0, True),
        (Wn1, bn1, Wr1, br1, g1, be1, rm1, rv1, True),
        (Wn2, bn2, Wr2, br2, g1, be1, rm1, rv1, False),
    ]
    for li, (Wn, bn_, Wr, br, g, be, rm, rv, with_bn) in enumerate(params):
        if li == 0:
            un, den = _agg_den_call(h, lsrc, ldst, lel, cnts)
            den_col = jnp.concatenate(
                [den[c].reshape(40 * D)[:HD] for c in range(NC)]
            ).reshape(N, 1)
        else:
            un, _ = _agg_call(h, lsrc, ldst, lel, cnts)
        un_flat = un.reshape(N, D)
        call = _layer_bn_call if with_bn else _layer_plain_call
        h = call(un_flat, den_col, h, Wn, r2(bn_), Wr, r2(br),
                 r2(g), r2(be), r2(rm), r2(rv))
    return h


# speedup vs baseline: 4.3142x; 1.0589x over previous
import functools

import jax
import jax.numpy as jnp
from jax import lax
from jax.experimental import pallas as pl
from jax.experimental.pallas import tpu as pltpu
from jax.experimental.pallas import tpu_sc as plsc

N = 10000
E = 320000
D = 128
TAU = 0.5

NC = 2
NS = 16
NW = NC * NS
K = 80
EPW = E // NW
NCH1 = EPW // K
EPS = E // NS
CP = 400
NCH2 = EPS // CP
HD = N // NC
SPR = 5056
DB = 5008
LCAP = EPS + 320
ZL = 104
RS = 312
BM = 400

_mesh = plsc.VectorSubcoreMesh(core_axis_name="c", subcore_axis_name="s")
_sc_params = pltpu.CompilerParams(needs_layout_passes=False)


def _ew_body(xn, src, dst, el_out,
             src_v0, dst_v0, a_v0, b_v0, el_v0,
             src_v1, dst_v1, a_v1, b_v1, el_v1,
             tile_v, ga0, gb0, es0, ga1, gb1, es1):
    c = lax.axis_index("c")
    s = lax.axis_index("s")
    wid = s * NC + c
    bufs = [(src_v0, dst_v0, a_v0, b_v0, el_v0, ga0, gb0, es0),
            (src_v1, dst_v1, a_v1, b_v1, el_v1, ga1, gb1, es1)]

    def prefetch(i, b):
        sv, dv, av, bv, _, ga, gb, _ = bufs[b]
        base = wid * EPW + i * K
        pltpu.sync_copy(src.at[pl.ds(base, K)], sv)
        pltpu.sync_copy(dst.at[pl.ds(base, K)], dv)
        pltpu.async_copy(xn.at[sv], av, ga)
        pltpu.async_copy(xn.at[dv], bv, gb)

    def wait_gathers(b):
        sv, dv, av, bv, ga, gb = (bufs[b][0], bufs[b][1], bufs[b][2],
                                  bufs[b][3], bufs[b][5], bufs[b][6])
        pltpu.make_async_copy(xn.at[sv], av, ga).wait()
        pltpu.make_async_copy(xn.at[dv], bv, gb).wait()

    def compute_export(i, b):
        _, _, av, bv, ev, _, _, es = bufs[b]
        ioff = lax.iota(jnp.int32, 16) * 16
        for g in range(K // 16):
            for e2 in range(16):
                e = g * 16 + e2
                acc = av[e, pl.ds(0, 16)] * bv[e, pl.ds(0, 16)]
                for j in range(1, 8):
                    acc = acc + (av[e, pl.ds(16 * j, 16)]
                                 * bv[e, pl.ds(16 * j, 16)])
                tile_v[pl.ds(e2 * 16, 16)] = acc
            sims = plsc.load_gather(tile_v, [ioff])
            for l in range(1, 16):
                sims = sims + plsc.load_gather(tile_v, [ioff + l])
            ev[pl.ds(16 * g, 16)] = jnp.exp(sims * (1.0 / TAU))
        base = wid * EPW + i * K
        pltpu.async_copy(ev, el_out.at[pl.ds(base, K)], es)

    def wait_export(b):
        ev, es = bufs[b][4], bufs[b][7]
        pltpu.make_async_copy(ev, el_out.at[pl.ds(wid * EPW, K)], es).wait()

    prefetch(0, 0)

    def pair(p, _):
        i0 = 2 * p
        prefetch(i0 + 1, 1)
        wait_gathers(0)

        @pl.when(p > 0)
        def _():
            wait_export(0)
        compute_export(i0, 0)

        @pl.when(i0 + 2 < NCH1)
        def _():
            prefetch(i0 + 2, 0)
        wait_gathers(1)

        @pl.when(p > 0)
        def _():
            wait_export(1)
        compute_export(i0 + 1, 1)
        return 0
    lax.fori_loop(0, NCH1 // 2, pair, 0)

    wait_gathers(0)
    wait_export(0)
    compute_export(NCH1 - 1, 0)
    wait_export(1)
    wait_export(0)


_ew_call = pl.kernel(
    _ew_body,
    out_type=jax.ShapeDtypeStruct((E,), jnp.float32),
    mesh=_mesh,
    compiler_params=_sc_params,
    scratch_types=[
        pltpu.VMEM((K,), jnp.int32),
        pltpu.VMEM((K,), jnp.int32),
        pltpu.VMEM((K, D), jnp.float32),
        pltpu.VMEM((K, D), jnp.float32),
        pltpu.VMEM((K,), jnp.float32),
        pltpu.VMEM((K,), jnp.int32),
        pltpu.VMEM((K,), jnp.int32),
        pltpu.VMEM((K, D), jnp.float32),
        pltpu.VMEM((K, D), jnp.float32),
        pltpu.VMEM((K,), jnp.float32),
        pltpu.VMEM((256,), jnp.float32),
        pltpu.SemaphoreType.DMA,
        pltpu.SemaphoreType.DMA,
        pltpu.SemaphoreType.DMA,
        pltpu.SemaphoreType.DMA,
        pltpu.SemaphoreType.DMA,
        pltpu.SemaphoreType.DMA,
    ],
)


def _part_body(src, dst, el, lsrc, ldst, lel, cnts,
               src_v0, dst_v0, el_v0, src_v1, dst_v1, el_v1,
               ssrc, sdst, sel, cnt_v, ls0, ls1):
    c = lax.axis_index("c")
    s = lax.axis_index("s")
    lo = c * HD
    bufs = [(src_v0, dst_v0, el_v0, ls0), (src_v1, dst_v1, el_v1, ls1)]

    def prefetch(i, b):
        sv, dv, ev, ls = bufs[b]
        base = s * EPS + i * CP
        pltpu.async_copy(src.at[pl.ds(base, CP)], sv, ls)
        pltpu.async_copy(dst.at[pl.ds(base, CP)], dv, ls)
        pltpu.async_copy(el.at[pl.ds(base, CP)], ev, ls)

    def wait_loads(i, b):
        sv, dv, ev, ls = bufs[b]
        base = s * EPS + i * CP
        pltpu.make_async_copy(src.at[pl.ds(base, CP)], sv, ls).wait()
        pltpu.make_async_copy(dst.at[pl.ds(base, CP)], dv, ls).wait()
        pltpu.make_async_copy(el.at[pl.ds(base, CP)], ev, ls).wait()

    def compact(b, wp):
        sv, dv, ev, _ = bufs[b]
        for g in range(CP // 16):
            sl = pl.ds(g * 16, 16)
            d16 = dv[sl]
            mask = (d16 >= lo) & (d16 < lo + HD)
            plsc.store_compressed(ssrc.at[pl.ds(wp, 16)], sv[sl], mask=mask)
            plsc.store_compressed(sdst.at[pl.ds(wp, 16)], d16 - lo, mask=mask)
            plsc.store_compressed(sel.at[pl.ds(wp, 16)], ev[sl], mask=mask)
            wp = wp + plsc.all_reduce_population_count(mask)[0]
        return wp

    prefetch(0, 0)

    def pair(p, wp):
        i0 = 2 * p
        prefetch(i0 + 1, 1)
        wait_loads(i0, 0)
        wp = compact(0, wp)

        @pl.when(i0 + 2 < NCH2)
        def _():
            prefetch(i0 + 2, 0)
        wait_loads(i0 + 1, 1)
        return compact(1, wp)
    wp = lax.fori_loop(0, NCH2 // 2, pair, jnp.int32(0))

    def padz(i, _):
        off = pl.ds(wp + i * 16, 16)
        ssrc[off] = jnp.zeros((16,), jnp.int32)
        sdst[off] = jnp.zeros((16,), jnp.int32)
        sel[off] = jnp.zeros((16,), jnp.float32)
        return 0
    lax.fori_loop(0, 20, padz, 0)

    rbase = (c * NS + s) * LCAP
    pltpu.sync_copy(ssrc, lsrc.at[pl.ds(rbase, LCAP)])
    pltpu.sync_copy(sdst, ldst.at[pl.ds(rbase, LCAP)])
    pltpu.sync_copy(sel, lel.at[pl.ds(rbase, LCAP)])
    cnt_v[:] = jnp.full((16,), wp, jnp.int32)
    pltpu.sync_copy(cnt_v, cnts.at[pl.ds((c * NS + s) * 16, 16)])


_part_call = pl.kernel(
    _part_body,
    out_type=(jax.ShapeDtypeStruct((NC * NS * LCAP,), jnp.int32),
              jax.ShapeDtypeStruct((NC * NS * LCAP,), jnp.int32),
              jax.ShapeDtypeStruct((NC * NS * LCAP,), jnp.float32),
              jax.ShapeDtypeStruct((NC * NS * 16,), jnp.int32)),
    mesh=_mesh,
    compiler_params=_sc_params,
    scratch_types=[
        pltpu.VMEM((CP,), jnp.int32),
        pltpu.VMEM((CP,), jnp.int32),
        pltpu.VMEM((CP,), jnp.float32),
        pltpu.VMEM((CP,), jnp.int32),
        pltpu.VMEM((CP,), jnp.int32),
        pltpu.VMEM((CP,), jnp.float32),
        pltpu.VMEM((LCAP,), jnp.int32),
        pltpu.VMEM((LCAP,), jnp.int32),
        pltpu.VMEM((LCAP,), jnp.float32),
        pltpu.VMEM((16,), jnp.int32),
        pltpu.SemaphoreType.DMA,
        pltpu.SemaphoreType.DMA,
    ],
)


def _agg_body(h, lsrc, ldst, lel, cnts, un_out, den_out,
              src_v0, dst_v0, el_v0, rows_v0, gs0, ss0,
              src_v1, dst_v1, el_v1, rows_v1, gs1, ss1,
              src_v2, dst_v2, el_v2, rows_v2, gs2, ss2,
              zbuf_v, cnts_v, den_v, didx_v, agg_sp, *, with_den):
    c = lax.axis_index("c")
    s = lax.axis_index("s")
    bufs = [(src_v0, dst_v0, el_v0, rows_v0, gs0, ss0),
            (src_v1, dst_v1, el_v1, rows_v1, gs1, ss1),
            (src_v2, dst_v2, el_v2, rows_v2, gs2, ss2)]

    def zf(r, _):
        for j in range(8):
            zbuf_v[r, pl.ds(16 * j, 16)] = jnp.zeros((16,), jnp.float32)
        return 0
    lax.fori_loop(0, ZL, zf, 0)
    for t in range(RS // ZL):
        pltpu.sync_copy(zbuf_v, agg_sp.at[pl.ds(s * RS + t * ZL, ZL)])

    @pl.when(s == NS - 1)
    def _():
        pltpu.sync_copy(zbuf_v.at[pl.ds(0, SPR - NS * RS)],
                        agg_sp.at[pl.ds(NS * RS, SPR - NS * RS)])

    if with_den:
        def zd(r, _):
            for j in range(8):
                den_v[r, pl.ds(16 * j, 16)] = jnp.zeros((16,), jnp.float32)
            return 0
        lax.fori_loop(0, 48, zd, 0)
        ioz = lax.iota(jnp.int32, 16)
        for g in range(3):
            didx_v[pl.ds(g * 16, 16)] = ioz + (DB + g * 16)
    plsc.subcore_barrier()

    pltpu.sync_copy(cnts.at[pl.ds((c * NS + s) * 16, 16)], cnts_v)
    cnt = cnts_v[pl.ds(0, 16)][0]
    ntr = (cnt + (3 * K - 1)) // (3 * K)
    rbase = (c * NS + s) * LCAP

    def prefetch(j, b):
        sv, dv, ev, rv, gs, _ = bufs[b]
        base = rbase + j * K
        pltpu.sync_copy(lsrc.at[pl.ds(base, K)], sv)
        pltpu.sync_copy(ldst.at[pl.ds(base, K)], dv)
        pltpu.sync_copy(lel.at[pl.ds(base, K)], ev)
        pltpu.async_copy(h.at[sv], rv, gs)

    def wait_gather(b):
        sv, rv, gs = bufs[b][0], bufs[b][3], bufs[b][4]
        pltpu.make_async_copy(h.at[sv], rv, gs).wait()

    def wait_scatter(b):
        dv, rv, ss = bufs[b][1], bufs[b][3], bufs[b][5]
        pltpu.make_async_copy(rv, agg_sp.at[dv], ss).wait()

    def compute_scatter(b):
        sv, dv, ev, rv, _, ss = bufs[b]
        ioz16 = lax.iota(jnp.int32, 16)
        for g in range(K // 16):
            el16 = ev[pl.ds(g * 16, 16)]
            if with_den:
                d16 = dv[pl.ds(g * 16, 16)]
            for e2 in range(16):
                e = g * 16 + e2
                wv = jnp.full((16,), el16[e2], jnp.float32)
                for j in range(8):
                    rv[e, pl.ds(16 * j, 16)] = rv[e, pl.ds(16 * j, 16)] * wv
                if with_den:
                    ld = d16[e2]
                    row = lax.shift_right_logical(ld, 7)
                    off = (lax.shift_right_logical(ld, 4) & 7) * 16
                    lane = ld & 15
                    m = ioz16 == jnp.full((16,), lane, jnp.int32)
                    addv = jnp.where(m, wv, jnp.zeros((16,), jnp.float32))
                    den_v[row, pl.ds(off, 16)] = (
                        den_v[row, pl.ds(off, 16)] + addv)
        pltpu.async_copy(rv, agg_sp.at[dv], ss, add=True)

    prefetch(0, 0)

    def triple(t, _):
        j0 = 3 * t
        wait_gather(0)

        @pl.when(t > 0)
        def _():
            wait_scatter(1)
        prefetch(j0 + 1, 1)
        compute_scatter(0)

        wait_gather(1)

        @pl.when(t > 0)
        def _():
            wait_scatter(2)
        prefetch(j0 + 2, 2)
        compute_scatter(1)

        wait_gather(2)
        wait_scatter(0)
        prefetch(j0 + 3, 0)
        compute_scatter(2)
        return 0
    lax.fori_loop(0, ntr, triple, 0)

    wait_gather(0)

    @pl.when(ntr > 0)
    def _():
        wait_scatter(1)
        wait_scatter(2)

    if with_den:
        pltpu.sync_copy(den_v, agg_sp.at[didx_v], add=True)

    plsc.subcore_barrier()
    for t in range(RS // ZL):
        sl = pl.ds(s * RS + t * ZL, ZL)
        pltpu.sync_copy(agg_sp.at[sl], zbuf_v)
        pltpu.sync_copy(zbuf_v, un_out.at[c, sl])

    @pl.when(s == NS - 1)
    def _():
        tl = pl.ds(NS * RS, HD - NS * RS)
        pltpu.sync_copy(agg_sp.at[tl], zbuf_v.at[pl.ds(0, HD - NS * RS)])
        pltpu.sync_copy(zbuf_v.at[pl.ds(0, HD - NS * RS)], un_out.at[c, tl])

    if with_den:
        @pl.when(s == 0)
        def _():
            dl = pl.ds(DB, 40)
            pltpu.sync_copy(agg_sp.at[dl], zbuf_v.at[pl.ds(0, 40)])
            pltpu.sync_copy(zbuf_v.at[pl.ds(0, 40)], den_out.at[c])


def _make_agg_call(with_den):
    return pl.kernel(
        functools.partial(_agg_body, with_den=with_den),
        out_type=(jax.ShapeDtypeStruct((NC, HD, D), jnp.float32),
                  jax.ShapeDtypeStruct((NC, 40, D), jnp.float32)),
        mesh=_mesh,
        compiler_params=_sc_params,
        scratch_types=(
            [pltpu.VMEM((K,), jnp.int32),
             pltpu.VMEM((K,), jnp.int32),
             pltpu.VMEM((K,), jnp.float32),
             pltpu.VMEM((K, D), jnp.float32),
             pltpu.SemaphoreType.DMA,
             pltpu.SemaphoreType.DMA] * 3
            + [pltpu.VMEM((ZL, D), jnp.float32),
               pltpu.VMEM((16,), jnp.int32),
               pltpu.VMEM((48, D), jnp.float32),
               pltpu.VMEM((48,), jnp.int32),
               pltpu.VMEM_SHARED((SPR, D), jnp.float32)]
        ),
    )


_agg_den_call = _make_agg_call(True)
_agg_call = _make_agg_call(False)


def _norm_body(x_ref, o_ref):
    x = x_ref[...]
    n = jnp.sqrt(jnp.sum(x * x, axis=1, keepdims=True))
    o_ref[...] = x / (n + 1e-12)


_norm_call = pl.pallas_call(
    _norm_body,
    grid=(N // BM,),
    in_specs=[pl.BlockSpec((BM, D), lambda i: (i, 0))],
    out_specs=pl.BlockSpec((BM, D), lambda i: (i, 0)),
    out_shape=jax.ShapeDtypeStruct((N, D), jnp.float32),
)


def _layer_body(un_ref, den_ref, h_ref, wn_ref, bn_ref, wr_ref, br_ref,
                g_ref, be_ref, rm_ref, rv_ref, o_ref, *, with_bn):
    den = den_ref[...]
    agg = un_ref[...] / (den + 1e-16)
    h = h_ref[...]
    y = lax.dot_general(agg, wn_ref[...], (((1,), (1,)), ((), ())),
                        preferred_element_type=jnp.float32)
    y = y + lax.dot_general(h, wr_ref[...], (((1,), (1,)), ((), ())),
                            preferred_element_type=jnp.float32)
    y = y + bn_ref[...] + br_ref[...]
    if with_bn:
        y = (y - rm_ref[...]) * lax.rsqrt(rv_ref[...] + 1e-5) * g_ref[...]
        y = jnp.maximum(y + be_ref[...], 0.0)
    o_ref[...] = y


def _make_layer_call(with_bn):
    full = lambda i: (0, 0)
    return pl.pallas_call(
        functools.partial(_layer_body, with_bn=with_bn),
        grid=(N // BM,),
        in_specs=[
            pl.BlockSpec((BM, D), lambda i: (i, 0)),
            pl.BlockSpec((BM, 1), lambda i: (i, 0)),
            pl.BlockSpec((BM, D), lambda i: (i, 0)),
            pl.BlockSpec((D, D), full),
            pl.BlockSpec((1, D), full),
            pl.BlockSpec((D, D), full),
            pl.BlockSpec((1, D), full),
            pl.BlockSpec((1, D), full),
            pl.BlockSpec((1, D), full),
            pl.BlockSpec((1, D), full),
            pl.BlockSpec((1, D), full),
        ],
        out_specs=pl.BlockSpec((BM, D), lambda i: (i, 0)),
        out_shape=jax.ShapeDtypeStruct((N, D), jnp.float32),
    )


_layer_bn_call = _make_layer_call(True)
_layer_plain_call = _make_layer_call(False)


def kernel(x, edge_index, Wn0, bn0, Wr0, br0, Wn1, bn1, Wr1, br1,
           Wn2, bn2, Wr2, br2, g0, be0, rm0, rv0, g1, be1, rm1, rv1):
    src = edge_index[0]
    dst = edge_index[1]
    xn = _norm_call(x)
    el = _ew_call(xn, src, dst)
    lsrc, ldst, lel, cnts = _part_call(src, dst, el)

    r2 = lambda v: v.reshape(1, D)
    h = x
    den_col = None
    params = [
        (Wn0, bn0, Wr0, br0, g0, be0, rm0, rv0, True),
        (Wn1, bn1, Wr1, br1, g1, be1, rm1, rv1, True),
        (Wn2, bn2, Wr2, br2, g1, be1, rm1, rv1, False),
    ]
    for li, (Wn, bn_, Wr, br, g, be, rm, rv, with_bn) in enumerate(params):
        if li == 0:
            un, den = _agg_den_call(h, lsrc, ldst, lel, cnts)
            den_col = jnp.concatenate(
                [den[c].reshape(40 * D)[:HD] for c in range(NC)]
            ).reshape(N, 1)
        else:
            un, _ = _agg_call(h, lsrc, ldst, lel, cnts)
        un_flat = un.reshape(N, D)
        call = _layer_bn_call if with_bn else _layer_plain_call
        h = call(un_flat, den_col, h, Wn, r2(bn_), Wr, r2(br),
                 r2(g), r2(be), r2(rm), r2(rv))
    return h


# speedup vs baseline: 4.9026x; 1.1364x over previous
import functools

import jax
import jax.numpy as jnp
from jax import lax
from jax.experimental import pallas as pl
from jax.experimental.pallas import tpu as pltpu
from jax.experimental.pallas import tpu_sc as plsc

N = 10000
E = 320000
D = 128
TAU = 0.5

NC = 2
NS = 16
NW = NC * NS
K = 80
EPW = E // NW
NCH1 = EPW // K
EPS = E // NS
CP = 400
NCH2 = EPS // CP
HD = N // NC
SPR = 5056
DB = 5008
LCAP = EPS + 560
ZL = 104
RS = 312
BM = 400

_mesh = plsc.VectorSubcoreMesh(core_axis_name="c", subcore_axis_name="s")
_sc_params = pltpu.CompilerParams(needs_layout_passes=False)


def _ew_body(xn, src, dst, el_out,
             sidx_v, didx_v,
             a_v0, b_v0, el_v0, a_v1, b_v1, el_v1,
             tile_v, ga0, gb0, es0, ga1, gb1, es1):
    c = lax.axis_index("c")
    s = lax.axis_index("s")
    wid = s * NC + c
    bufs = [(a_v0, b_v0, el_v0, ga0, gb0, es0),
            (a_v1, b_v1, el_v1, ga1, gb1, es1)]

    pltpu.sync_copy(src.at[pl.ds(wid * EPW, EPW)], sidx_v)
    pltpu.sync_copy(dst.at[pl.ds(wid * EPW, EPW)], didx_v)

    def prefetch(i, b):
        av, bv, _, ga, gb, _ = bufs[b]
        pltpu.async_copy(xn.at[sidx_v.at[pl.ds(i * K, K)]], av, ga)
        pltpu.async_copy(xn.at[didx_v.at[pl.ds(i * K, K)]], bv, gb)

    def wait_gathers(i, b):
        av, bv, ga, gb = bufs[b][0], bufs[b][1], bufs[b][3], bufs[b][4]
        pltpu.make_async_copy(xn.at[sidx_v.at[pl.ds(i * K, K)]], av,
                              ga).wait()
        pltpu.make_async_copy(xn.at[didx_v.at[pl.ds(i * K, K)]], bv,
                              gb).wait()

    def compute_export(i, b):
        av, bv, ev, _, _, es = bufs[b]
        ioff = lax.iota(jnp.int32, 16) * 16
        for g in range(K // 16):
            for e2 in range(16):
                e = g * 16 + e2
                acc = av[e, pl.ds(0, 16)] * bv[e, pl.ds(0, 16)]
                for j in range(1, 8):
                    acc = acc + (av[e, pl.ds(16 * j, 16)]
                                 * bv[e, pl.ds(16 * j, 16)])
                tile_v[pl.ds(e2 * 16, 16)] = acc
            sims = plsc.load_gather(tile_v, [ioff])
            for l in range(1, 16):
                sims = sims + plsc.load_gather(tile_v, [ioff + l])
            ev[pl.ds(16 * g, 16)] = jnp.exp(sims * (1.0 / TAU))
        base = wid * EPW + i * K
        pltpu.async_copy(ev, el_out.at[pl.ds(base, K)], es)

    def wait_export(b):
        ev, es = bufs[b][2], bufs[b][5]
        pltpu.make_async_copy(ev, el_out.at[pl.ds(wid * EPW, K)], es).wait()

    prefetch(0, 0)

    def pair(p, _):
        i0 = 2 * p
        prefetch(i0 + 1, 1)
        wait_gathers(i0, 0)

        @pl.when(p > 0)
        def _():
            wait_export(0)
        compute_export(i0, 0)

        @pl.when(i0 + 2 < NCH1)
        def _():
            prefetch(i0 + 2, 0)
        wait_gathers(i0 + 1, 1)

        @pl.when(p > 0)
        def _():
            wait_export(1)
        compute_export(i0 + 1, 1)
        return 0
    lax.fori_loop(0, NCH1 // 2, pair, 0)

    wait_gathers(NCH1 - 1, 0)
    wait_export(0)
    compute_export(NCH1 - 1, 0)
    wait_export(1)
    wait_export(0)


_ew_call = pl.kernel(
    _ew_body,
    out_type=jax.ShapeDtypeStruct((E,), jnp.float32),
    mesh=_mesh,
    compiler_params=_sc_params,
    scratch_types=[
        pltpu.VMEM((EPW,), jnp.int32),
        pltpu.VMEM((EPW,), jnp.int32),
        pltpu.VMEM((K, D), jnp.float32),
        pltpu.VMEM((K, D), jnp.float32),
        pltpu.VMEM((K,), jnp.float32),
        pltpu.VMEM((K, D), jnp.float32),
        pltpu.VMEM((K, D), jnp.float32),
        pltpu.VMEM((K,), jnp.float32),
        pltpu.VMEM((256,), jnp.float32),
        pltpu.SemaphoreType.DMA,
        pltpu.SemaphoreType.DMA,
        pltpu.SemaphoreType.DMA,
        pltpu.SemaphoreType.DMA,
        pltpu.SemaphoreType.DMA,
        pltpu.SemaphoreType.DMA,
    ],
)


def _part_body(src, dst, el, lsrc, ldst, lel, cnts,
               src_v0, dst_v0, el_v0, src_v1, dst_v1, el_v1,
               ssrc, sdst, sel, cnt_v, ls0, ls1):
    c = lax.axis_index("c")
    s = lax.axis_index("s")
    lo = c * HD
    bufs = [(src_v0, dst_v0, el_v0, ls0), (src_v1, dst_v1, el_v1, ls1)]

    def prefetch(i, b):
        sv, dv, ev, ls = bufs[b]
        base = s * EPS + i * CP
        pltpu.async_copy(src.at[pl.ds(base, CP)], sv, ls)
        pltpu.async_copy(dst.at[pl.ds(base, CP)], dv, ls)
        pltpu.async_copy(el.at[pl.ds(base, CP)], ev, ls)

    def wait_loads(i, b):
        sv, dv, ev, ls = bufs[b]
        base = s * EPS + i * CP
        pltpu.make_async_copy(src.at[pl.ds(base, CP)], sv, ls).wait()
        pltpu.make_async_copy(dst.at[pl.ds(base, CP)], dv, ls).wait()
        pltpu.make_async_copy(el.at[pl.ds(base, CP)], ev, ls).wait()

    def compact(b, wp):
        sv, dv, ev, _ = bufs[b]
        for g in range(CP // 16):
            sl = pl.ds(g * 16, 16)
            d16 = dv[sl]
            mask = (d16 >= lo) & (d16 < lo + HD)
            plsc.store_compressed(ssrc.at[pl.ds(wp, 16)], sv[sl], mask=mask)
            plsc.store_compressed(sdst.at[pl.ds(wp, 16)], d16 - lo, mask=mask)
            plsc.store_compressed(sel.at[pl.ds(wp, 16)], ev[sl], mask=mask)
            wp = wp + plsc.all_reduce_population_count(mask)[0]
        return wp

    prefetch(0, 0)

    def pair(p, wp):
        i0 = 2 * p
        prefetch(i0 + 1, 1)
        wait_loads(i0, 0)
        wp = compact(0, wp)

        @pl.when(i0 + 2 < NCH2)
        def _():
            prefetch(i0 + 2, 0)
        wait_loads(i0 + 1, 1)
        return compact(1, wp)
    wp = lax.fori_loop(0, NCH2 // 2, pair, jnp.int32(0))

    def padz(i, _):
        off = pl.ds(wp + i * 16, 16)
        ssrc[off] = jnp.zeros((16,), jnp.int32)
        sdst[off] = jnp.zeros((16,), jnp.int32)
        sel[off] = jnp.zeros((16,), jnp.float32)
        return 0
    lax.fori_loop(0, 35, padz, 0)

    rbase = (c * NS + s) * LCAP
    pltpu.sync_copy(ssrc, lsrc.at[pl.ds(rbase, LCAP)])
    pltpu.sync_copy(sdst, ldst.at[pl.ds(rbase, LCAP)])
    pltpu.sync_copy(sel, lel.at[pl.ds(rbase, LCAP)])
    cnt_v[:] = jnp.full((16,), wp, jnp.int32)
    pltpu.sync_copy(cnt_v, cnts.at[pl.ds((c * NS + s) * 16, 16)])


_part_call = pl.kernel(
    _part_body,
    out_type=(jax.ShapeDtypeStruct((NC * NS * LCAP,), jnp.int32),
              jax.ShapeDtypeStruct((NC * NS * LCAP,), jnp.int32),
              jax.ShapeDtypeStruct((NC * NS * LCAP,), jnp.float32),
              jax.ShapeDtypeStruct((NC * NS * 16,), jnp.int32)),
    mesh=_mesh,
    compiler_params=_sc_params,
    scratch_types=[
        pltpu.VMEM((CP,), jnp.int32),
        pltpu.VMEM((CP,), jnp.int32),
        pltpu.VMEM((CP,), jnp.float32),
        pltpu.VMEM((CP,), jnp.int32),
        pltpu.VMEM((CP,), jnp.int32),
        pltpu.VMEM((CP,), jnp.float32),
        pltpu.VMEM((LCAP,), jnp.int32),
        pltpu.VMEM((LCAP,), jnp.int32),
        pltpu.VMEM((LCAP,), jnp.float32),
        pltpu.VMEM((16,), jnp.int32),
        pltpu.SemaphoreType.DMA,
        pltpu.SemaphoreType.DMA,
    ],
)


def _agg_body(h, lsrc, ldst, lel, cnts, un_out, den_out,
              sv0, dv0, ev0, ds0, rows0, ls0, gs0, ss0,
              sv1, dv1, ev1, ds1, rows1, ls1, gs1, ss1,
              sv2, dv2, ev2, ds2, rows2, ls2, gs2, ss2,
              sv3, dv3, ev3, ds3, rows3, ls3, gs3, ss3,
              zbuf_v, cnts_v, den_v, didx_v, agg_sp, *, with_den):
    c = lax.axis_index("c")
    s = lax.axis_index("s")
    bufs = [(sv0, dv0, ev0, ds0, rows0, ls0, gs0, ss0),
            (sv1, dv1, ev1, ds1, rows1, ls1, gs1, ss1),
            (sv2, dv2, ev2, ds2, rows2, ls2, gs2, ss2),
            (sv3, dv3, ev3, ds3, rows3, ls3, gs3, ss3)]

    def zf(r, _):
        for j in range(8):
            zbuf_v[r, pl.ds(16 * j, 16)] = jnp.zeros((16,), jnp.float32)
        return 0
    lax.fori_loop(0, ZL, zf, 0)
    for t in range(RS // ZL):
        pltpu.sync_copy(zbuf_v, agg_sp.at[pl.ds(s * RS + t * ZL, ZL)])

    @pl.when(s == NS - 1)
    def _():
        pltpu.sync_copy(zbuf_v.at[pl.ds(0, SPR - NS * RS)],
                        agg_sp.at[pl.ds(NS * RS, SPR - NS * RS)])

    if with_den:
        def zd(r, _):
            for j in range(8):
                den_v[r, pl.ds(16 * j, 16)] = jnp.zeros((16,), jnp.float32)
            return 0
        lax.fori_loop(0, 48, zd, 0)
        ioz = lax.iota(jnp.int32, 16)
        for g in range(3):
            didx_v[pl.ds(g * 16, 16)] = ioz + (DB + g * 16)
    plsc.subcore_barrier()

    pltpu.sync_copy(cnts.at[pl.ds((c * NS + s) * 16, 16)], cnts_v)
    cnt = cnts_v[pl.ds(0, 16)][0]
    nq = (cnt + (4 * K - 1)) // (4 * K)
    rbase = (c * NS + s) * LCAP

    def loads(j, b):
        sv, dv, ev, ls = bufs[b][0], bufs[b][1], bufs[b][2], bufs[b][5]
        base = rbase + j * K
        pltpu.async_copy(lsrc.at[pl.ds(base, K)], sv, ls)
        pltpu.async_copy(ldst.at[pl.ds(base, K)], dv, ls)
        pltpu.async_copy(lel.at[pl.ds(base, K)], ev, ls)

    def wait_loads(j, b):
        sv, dv, ev, ls = bufs[b][0], bufs[b][1], bufs[b][2], bufs[b][5]
        base = rbase + j * K
        pltpu.make_async_copy(lsrc.at[pl.ds(base, K)], sv, ls).wait()
        pltpu.make_async_copy(ldst.at[pl.ds(base, K)], dv, ls).wait()
        pltpu.make_async_copy(lel.at[pl.ds(base, K)], ev, ls).wait()

    def gather(b):
        sv, rv, gs = bufs[b][0], bufs[b][4], bufs[b][6]
        pltpu.async_copy(h.at[sv], rv, gs)

    def wait_gather(b):
        sv, rv, gs = bufs[b][0], bufs[b][4], bufs[b][6]
        pltpu.make_async_copy(h.at[sv], rv, gs).wait()

    def wait_scatter(b):
        dsv, rv, ss = bufs[b][3], bufs[b][4], bufs[b][7]
        pltpu.make_async_copy(rv, agg_sp.at[dsv], ss).wait()

    def compute_scatter(b):
        dv, ev, dsv, rv, ss = (bufs[b][1], bufs[b][2], bufs[b][3],
                               bufs[b][4], bufs[b][7])
        ioz16 = lax.iota(jnp.int32, 16)

        def grp2(g, _):
            g16 = g * 16
            el16 = ev[pl.ds(g16, 16)]
            d16 = dv[pl.ds(g16, 16)]
            dsv[pl.ds(g16, 16)] = d16
            for e2 in range(16):
                e = g16 + e2
                wv = jnp.full((16,), el16[e2], jnp.float32)
                for j8 in range(8):
                    rv[e, pl.ds(16 * j8, 16)] = rv[e, pl.ds(16 * j8, 16)] * wv
                if with_den:
                    ld = d16[e2]
                    row = lax.shift_right_logical(ld, 7)
                    off = (lax.shift_right_logical(ld, 4) & 7) * 16
                    lane = ld & 15
                    m = ioz16 == jnp.full((16,), lane, jnp.int32)
                    addv = jnp.where(m, wv, jnp.zeros((16,), jnp.float32))
                    den_v[row, pl.ds(off, 16)] = (
                        den_v[row, pl.ds(off, 16)] + addv)
            return 0
        lax.fori_loop(0, K // 16, grp2, 0)
        pltpu.async_copy(rv, agg_sp.at[dsv], ss, add=True)

    loads(0, 0)
    loads(1, 1)
    loads(2, 2)
    wait_loads(0, 0)
    gather(0)
    wait_loads(1, 1)
    gather(1)

    def quad(t, _):
        for u in range(4):
            j = 4 * t + u
            b0, b2, b3 = u % 4, (u + 2) % 4, (u + 3) % 4
            if u < 2:
                @pl.when(t > 0)
                def _(b2=b2):
                    wait_scatter(b2)
            else:
                wait_scatter(b2)
            wait_loads(j + 2, b2)
            gather(b2)
            loads(j + 3, b3)
            wait_gather(b0)
            compute_scatter(b0)
        return 0
    lax.fori_loop(0, nq, quad, 0)

    fin = 4 * nq
    wait_loads(fin + 2, 2)
    wait_gather(0)
    wait_gather(1)

    @pl.when(nq > 0)
    def _():
        wait_scatter(2)
        wait_scatter(3)

    if with_den:
        pltpu.sync_copy(den_v, agg_sp.at[didx_v], add=True)

    plsc.subcore_barrier()
    for t in range(RS // ZL):
        sl = pl.ds(s * RS + t * ZL, ZL)
        pltpu.sync_copy(agg_sp.at[sl], zbuf_v)
        pltpu.sync_copy(zbuf_v, un_out.at[c, sl])

    @pl.when(s == NS - 1)
    def _():
        tl = pl.ds(NS * RS, HD - NS * RS)
        pltpu.sync_copy(agg_sp.at[tl], zbuf_v.at[pl.ds(0, HD - NS * RS)])
        pltpu.sync_copy(zbuf_v.at[pl.ds(0, HD - NS * RS)], un_out.at[c, tl])

    if with_den:
        @pl.when(s == 0)
        def _():
            dl = pl.ds(DB, 40)
            pltpu.sync_copy(agg_sp.at[dl], zbuf_v.at[pl.ds(0, 40)])
            pltpu.sync_copy(zbuf_v.at[pl.ds(0, 40)], den_out.at[c])


def _make_agg_call(with_den):
    return pl.kernel(
        functools.partial(_agg_body, with_den=with_den),
        out_type=(jax.ShapeDtypeStruct((NC, HD, D), jnp.float32),
                  jax.ShapeDtypeStruct((NC, 40, D), jnp.float32)),
        mesh=_mesh,
        compiler_params=_sc_params,
        scratch_types=(
            [pltpu.VMEM((K,), jnp.int32),
             pltpu.VMEM((K,), jnp.int32),
             pltpu.VMEM((K,), jnp.float32),
             pltpu.VMEM((K,), jnp.int32),
             pltpu.VMEM((K, D), jnp.float32),
             pltpu.SemaphoreType.DMA,
             pltpu.SemaphoreType.DMA,
             pltpu.SemaphoreType.DMA] * 4
            + [pltpu.VMEM((ZL, D), jnp.float32),
               pltpu.VMEM((16,), jnp.int32),
               pltpu.VMEM((48, D), jnp.float32),
               pltpu.VMEM((48,), jnp.int32),
               pltpu.VMEM_SHARED((SPR, D), jnp.float32)]
        ),
    )


_agg_den_call = _make_agg_call(True)
_agg_call = _make_agg_call(False)


def _norm_body(x_ref, o_ref):
    x = x_ref[...]
    n = jnp.sqrt(jnp.sum(x * x, axis=1, keepdims=True))
    o_ref[...] = x / (n + 1e-12)


_norm_call = pl.pallas_call(
    _norm_body,
    grid=(N // BM,),
    in_specs=[pl.BlockSpec((BM, D), lambda i: (i, 0))],
    out_specs=pl.BlockSpec((BM, D), lambda i: (i, 0)),
    out_shape=jax.ShapeDtypeStruct((N, D), jnp.float32),
)


def _layer_body(un_ref, den_ref, h_ref, wn_ref, bn_ref, wr_ref, br_ref,
                g_ref, be_ref, rm_ref, rv_ref, o_ref, *, with_bn):
    den = den_ref[...]
    agg = un_ref[...] / (den + 1e-16)
    h = h_ref[...]
    y = lax.dot_general(agg, wn_ref[...], (((1,), (1,)), ((), ())),
                        preferred_element_type=jnp.float32)
    y = y + lax.dot_general(h, wr_ref[...], (((1,), (1,)), ((), ())),
                            preferred_element_type=jnp.float32)
    y = y + bn_ref[...] + br_ref[...]
    if with_bn:
        y = (y - rm_ref[...]) * lax.rsqrt(rv_ref[...] + 1e-5) * g_ref[...]
        y = jnp.maximum(y + be_ref[...], 0.0)
    o_ref[...] = y


def _make_layer_call(with_bn):
    full = lambda i: (0, 0)
    return pl.pallas_call(
        functools.partial(_layer_body, with_bn=with_bn),
        grid=(N // BM,),
        in_specs=[
            pl.BlockSpec((BM, D), lambda i: (i, 0)),
            pl.BlockSpec((BM, 1), lambda i: (i, 0)),
            pl.BlockSpec((BM, D), lambda i: (i, 0)),
            pl.BlockSpec((D, D), full),
            pl.BlockSpec((1, D), full),
            pl.BlockSpec((D, D), full),
            pl.BlockSpec((1, D), full),
            pl.BlockSpec((1, D), full),
            pl.BlockSpec((1, D), full),
            pl.BlockSpec((1, D), full),
            pl.BlockSpec((1, D), full),
        ],
        out_specs=pl.BlockSpec((BM, D), lambda i: (i, 0)),
        out_shape=jax.ShapeDtypeStruct((N, D), jnp.float32),
    )


_layer_bn_call = _make_layer_call(True)
_layer_plain_call = _make_layer_call(False)


def kernel(x, edge_index, Wn0, bn0, Wr0, br0, Wn1, bn1, Wr1, br1,
           Wn2, bn2, Wr2, br2, g0, be0, rm0, rv0, g1, be1, rm1, rv1):
    src = edge_index[0]
    dst = edge_index[1]
    xn = _norm_call(x)
    el = _ew_call(xn, src, dst)
    lsrc, ldst, lel, cnts = _part_call(src, dst, el)

    r2 = lambda v: v.reshape(1, D)
    h = x
    den_col = None
    params = [
        (Wn0, bn0, Wr0, br0, g0, be0, rm0, rv0, True),
        (Wn1, bn1, Wr1, br1, g1, be1, rm1, rv1, True),
        (Wn2, bn2, Wr2, br2, g1, be1, rm1, rv1, False),
    ]
    for li, (Wn, bn_, Wr, br, g, be, rm, rv, with_bn) in enumerate(params):
        if li == 0:
            un, den = _agg_den_call(h, lsrc, ldst, lel, cnts)
            den_col = jnp.concatenate(
                [den[c].reshape(40 * D)[:HD] for c in range(NC)]
            ).reshape(N, 1)
        else:
            un, _ = _agg_call(h, lsrc, ldst, lel, cnts)
        un_flat = un.reshape(N, D)
        call = _layer_bn_call if with_bn else _layer_plain_call
        h = call(un_flat, den_col, h, Wn, r2(bn_), Wr, r2(br),
                 r2(g), r2(be), r2(rm), r2(rv))
    return h
